# baseline (speedup 1.0000x reference)
import functools

import jax
import jax.numpy as jnp
from jax import lax
from jax.experimental import pallas as pl
from jax.experimental.pallas import tpu as pltpu
from jax.experimental.pallas import tpu_sc as plsc

D_MODEL = 1024
INTER = 512
NUM_E = 8
N_TOK = 4096
BLK_T = 256
N_BLK = 40
XG_ROWS = N_BLK * BLK_T
SC_WORKERS = 32
CHUNK = 64


def _nt(a, b):
    return jax.lax.dot_general(
        a, b, (((1,), (1,)), ((), ())), preferred_element_type=jnp.float32
    )


def _cumsum0(x):
    n, l = x.shape
    s = 1
    while s < n:
        shifted = jnp.concatenate(
            [jnp.zeros((s, l), x.dtype), x[: n - s, :]], axis=0)
        x = x + shifted
        s *= 2
    return x


def _route_body(x_ref, rw_ref, rb_ref, sg_ref, su_ref, sd_ref,
                sh_ref, p_ref, eid_ref):
    xb = x_ref[...].astype(jnp.bfloat16)

    logits = _nt(xb, rw_ref[...].astype(jnp.bfloat16)) + rb_ref[...]
    iota = lax.broadcasted_iota(jnp.int32, logits.shape, 1)
    m0 = jnp.max(logits, axis=1, keepdims=True)
    i0 = jnp.min(jnp.where(logits == m0, iota, NUM_E), axis=1, keepdims=True)
    oh0 = iota == i0
    masked = jnp.where(oh0, -jnp.inf, logits)
    m1 = jnp.max(masked, axis=1, keepdims=True)
    i1 = jnp.min(jnp.where(masked == m1, iota, NUM_E), axis=1, keepdims=True)
    oh1 = iota == i1
    t = jnp.exp(m1 - m0)
    w0 = 1.0 / (1.0 + t)
    w1 = t / (1.0 + t)

    mask = (oh0 | oh1).astype(jnp.float32)
    cum = _cumsum0(mask)
    rank_excl = cum - mask
    counts = cum[N_TOK - 1:N_TOK, :]
    padded = jnp.floor((counts + (BLK_T - 1)) * (1.0 / BLK_T)) * BLK_T
    r8 = lax.broadcasted_iota(jnp.int32, (NUM_E, NUM_E), 0)
    c8 = lax.broadcasted_iota(jnp.int32, (NUM_E, NUM_E), 1)
    tri = (r8 < c8).astype(jnp.float32)
    base = jax.lax.dot_general(padded, tri, (((1,), (0,)), ((), ())),
                               preferred_element_type=jnp.float32)

    base0 = jnp.sum(jnp.where(oh0, base, 0.0), axis=1, keepdims=True)
    base1 = jnp.sum(jnp.where(oh1, base, 0.0), axis=1, keepdims=True)
    r0 = jnp.sum(jnp.where(oh0, rank_excl, 0.0), axis=1, keepdims=True)
    r1 = jnp.sum(jnp.where(oh1, rank_excl, 0.0), axis=1, keepdims=True)
    p0 = base0 + r0
    p1 = base1 + r1
    lane = lax.broadcasted_iota(jnp.int32, (N_TOK, NUM_E), 1)
    p_ref[...] = (jnp.where(lane == 0, p0, 0.0) + jnp.where(lane == 1, p1, 0.0)
                  + jnp.where(lane == 2, w0, 0.0) + jnp.where(lane == 3, w1, 0.0))

    eye = (r8 == c8).astype(jnp.float32)
    start_col = _nt(eye, base * (1.0 / BLK_T))
    i64 = lax.broadcasted_iota(jnp.int32, (NUM_E, 64), 1).astype(jnp.float32)
    eid = jnp.sum((start_col <= i64).astype(jnp.float32), axis=0,
                  keepdims=True) - 1.0
    eid_ref[...] = eid.astype(jnp.int32)

    sgb = sg_ref[...].astype(jnp.bfloat16)
    sub = su_ref[...].astype(jnp.bfloat16)
    sdb = sd_ref[...].astype(jnp.bfloat16)
    for c in range(4):
        sl = slice(c * (N_TOK // 4), (c + 1) * (N_TOK // 4))
        xc = xb[sl, :]
        g = _nt(xc, sgb)
        u = _nt(xc, sub)
        h = ((g * jax.nn.sigmoid(g)) * u).astype(jnp.bfloat16)
        sh_ref[sl, :] = _nt(h, sdb).astype(jnp.bfloat16)


def _route(flat, router_w, rb, sg, su, sd):
    return pl.pallas_call(
        _route_body,
        grid=(1,),
        in_specs=[
            pl.BlockSpec((N_TOK, D_MODEL), lambda i: (0, 0)),
            pl.BlockSpec((NUM_E, D_MODEL), lambda i: (0, 0)),
            pl.BlockSpec((1, NUM_E), lambda i: (0, 0)),
            pl.BlockSpec((INTER, D_MODEL), lambda i: (0, 0)),
            pl.BlockSpec((INTER, D_MODEL), lambda i: (0, 0)),
            pl.BlockSpec((D_MODEL, INTER), lambda i: (0, 0)),
        ],
        out_specs=[
            pl.BlockSpec((N_TOK, D_MODEL), lambda i: (0, 0)),
            pl.BlockSpec((N_TOK, NUM_E), lambda i: (0, 0)),
            pl.BlockSpec((1, 64), lambda i: (0, 0)),
        ],
        out_shape=[
            jax.ShapeDtypeStruct((N_TOK, D_MODEL), jnp.bfloat16),
            jax.ShapeDtypeStruct((N_TOK, NUM_E), jnp.float32),
            jax.ShapeDtypeStruct((1, 64), jnp.int32),
        ],
    )(flat, router_w, rb, sg, su, sd)


def _sc_scatter_body(x_hbm, p01_hbm, xg_hbm, idx_v, rows_v, sem):
    wid = lax.axis_index("s") * 2 + lax.axis_index("c")
    for j in range(4):
        abase = wid * 256 + j * CHUNK
        tbase = lax.rem(abase, N_TOK)
        pltpu.sync_copy(p01_hbm.at[pl.ds(abase, CHUNK)], idx_v)
        pltpu.sync_copy(x_hbm.at[pl.ds(tbase, CHUNK)], rows_v)
        pltpu.async_copy(rows_v, xg_hbm.at[idx_v], sem).wait()


def _sc_scatter(flat, p01):
    mesh = plsc.VectorSubcoreMesh(core_axis_name="c", subcore_axis_name="s")
    fn = functools.partial(
        pl.kernel,
        out_type=jax.ShapeDtypeStruct((XG_ROWS, D_MODEL), jnp.float32),
        mesh=mesh,
        scratch_types=[
            pltpu.VMEM((CHUNK,), jnp.int32),
            pltpu.VMEM((CHUNK, D_MODEL), jnp.float32),
            pltpu.SemaphoreType.DMA,
        ],
    )(_sc_scatter_body)
    return fn(flat, p01)


def _wcast_body(gw_ref, uw_ref, dw_ref, gb_ref, ub_ref, db_ref):
    gb_ref[...] = gw_ref[...].astype(jnp.bfloat16)
    ub_ref[...] = uw_ref[...].astype(jnp.bfloat16)
    db_ref[...] = dw_ref[...].astype(jnp.bfloat16)


def _wcast(gate_ws, up_ws, down_ws):
    return pl.pallas_call(
        _wcast_body,
        grid=(NUM_E,),
        in_specs=[
            pl.BlockSpec((1, INTER, D_MODEL), lambda e: (e, 0, 0)),
            pl.BlockSpec((1, INTER, D_MODEL), lambda e: (e, 0, 0)),
            pl.BlockSpec((1, D_MODEL, INTER), lambda e: (e, 0, 0)),
        ],
        out_specs=[
            pl.BlockSpec((1, INTER, D_MODEL), lambda e: (e, 0, 0)),
            pl.BlockSpec((1, INTER, D_MODEL), lambda e: (e, 0, 0)),
            pl.BlockSpec((1, D_MODEL, INTER), lambda e: (e, 0, 0)),
        ],
        out_shape=[
            jax.ShapeDtypeStruct((NUM_E, INTER, D_MODEL), jnp.bfloat16),
            jax.ShapeDtypeStruct((NUM_E, INTER, D_MODEL), jnp.bfloat16),
            jax.ShapeDtypeStruct((NUM_E, D_MODEL, INTER), jnp.bfloat16),
        ],
    )(gate_ws, up_ws, down_ws)


def _gemm_body(eid_ref, xg_ref, gw_ref, uw_ref, dw_ref, yg_ref):
    e = eid_ref[pl.program_id(0)]
    xb = xg_ref[...].astype(jnp.bfloat16)
    g = _nt(xb, gw_ref[e])
    u = _nt(xb, uw_ref[e])
    h = ((g * jax.nn.sigmoid(g)) * u).astype(jnp.bfloat16)
    yg_ref[...] = _nt(h, dw_ref[e])


def _grouped_gemm(xg, eid, gate_b, up_b, down_b):
    grid_spec = pltpu.PrefetchScalarGridSpec(
        num_scalar_prefetch=1,
        grid=(N_BLK,),
        in_specs=[
            pl.BlockSpec((BLK_T, D_MODEL), lambda i, eid: (i, 0)),
            pl.BlockSpec((NUM_E, INTER, D_MODEL), lambda i, eid: (0, 0, 0)),
            pl.BlockSpec((NUM_E, INTER, D_MODEL), lambda i, eid: (0, 0, 0)),
            pl.BlockSpec((NUM_E, D_MODEL, INTER), lambda i, eid: (0, 0, 0)),
        ],
        out_specs=pl.BlockSpec((BLK_T, D_MODEL), lambda i, eid: (i, 0)),
    )
    return pl.pallas_call(
        _gemm_body,
        grid_spec=grid_spec,
        out_shape=jax.ShapeDtypeStruct((XG_ROWS, D_MODEL), jnp.float32),
    )(eid, xg, gate_b, up_b, down_b)


def _sc_gather_body(yg_hbm, p01_hbm, a01_hbm, idx_v, rows_v, sem):
    wid = lax.axis_index("s") * 2 + lax.axis_index("c")
    for j in range(4):
        abase = wid * 256 + j * CHUNK
        pltpu.sync_copy(p01_hbm.at[pl.ds(abase, CHUNK)], idx_v)
        pltpu.async_copy(yg_hbm.at[idx_v], rows_v, sem).wait()
        pltpu.sync_copy(rows_v, a01_hbm.at[pl.ds(abase, CHUNK)])


def _sc_gather(yg, p01):
    mesh = plsc.VectorSubcoreMesh(core_axis_name="c", subcore_axis_name="s")
    fn = functools.partial(
        pl.kernel,
        out_type=jax.ShapeDtypeStruct((2 * N_TOK, D_MODEL), jnp.float32),
        mesh=mesh,
        scratch_types=[
            pltpu.VMEM((CHUNK,), jnp.int32),
            pltpu.VMEM((CHUNK, D_MODEL), jnp.float32),
            pltpu.SemaphoreType.DMA,
        ],
    )(_sc_gather_body)
    return fn(yg, p01)


def _combine_body(sh_ref, a0_ref, a1_ref, p_ref, out_ref):
    lane = lax.broadcasted_iota(jnp.int32, p_ref.shape, 1)
    w0 = jnp.sum(jnp.where(lane == 2, p_ref[...], 0.0), axis=1, keepdims=True)
    w1 = jnp.sum(jnp.where(lane == 3, p_ref[...], 0.0), axis=1, keepdims=True)
    out_ref[...] = (sh_ref[...].astype(jnp.float32)
                    + w0 * a0_ref[...] + w1 * a1_ref[...])


def _combine(sh, a01, p):
    tb = 1024
    return pl.pallas_call(
        _combine_body,
        grid=(N_TOK // tb,),
        in_specs=[
            pl.BlockSpec((tb, D_MODEL), lambda i: (i, 0)),
            pl.BlockSpec((tb, D_MODEL), lambda i: (i, 0)),
            pl.BlockSpec((tb, D_MODEL), lambda i: (i + N_TOK // tb, 0)),
            pl.BlockSpec((tb, NUM_E), lambda i: (i, 0)),
        ],
        out_specs=pl.BlockSpec((tb, D_MODEL), lambda i: (i, 0)),
        out_shape=jax.ShapeDtypeStruct((N_TOK, D_MODEL), jnp.float32),
    )(sh, a01, a01, p)


def kernel(x, router_w, router_bias, shared_gate_w, shared_up_w,
           shared_down_w, gate_ws, up_ws, down_ws):
    b, s, d = x.shape
    flat = x.reshape(b * s, d)
    rb = router_bias.reshape(1, NUM_E)

    sh, p, eid = _route(flat, router_w, rb, shared_gate_w,
                        shared_up_w, shared_down_w)
    p01 = jnp.concatenate([p[:, 0], p[:, 1]]).astype(jnp.int32)
    gate_b, up_b, down_b = _wcast(gate_ws, up_ws, down_ws)
    xg = _sc_scatter(flat, p01)
    yg = _grouped_gemm(xg, eid.reshape(64), gate_b, up_b, down_b)
    a01 = _sc_gather(yg, p01)
    out = _combine(sh, a01, p)
    return out.reshape(b, s, d)


# speedup vs baseline: 1.1055x; 1.1055x over previous
import functools

import jax
import jax.numpy as jnp
from jax import lax
from jax.experimental import pallas as pl
from jax.experimental.pallas import tpu as pltpu
from jax.experimental.pallas import tpu_sc as plsc

D_MODEL = 1024
INTER = 512
NUM_E = 8
N_TOK = 4096
BLK_T = 256
N_BLK = 40
XG_ROWS = N_BLK * BLK_T
SC_WORKERS = 32
CHUNK = 64


def _nt(a, b):
    return jax.lax.dot_general(
        a, b, (((1,), (1,)), ((), ())), preferred_element_type=jnp.float32
    )


def _cumsum0(x):
    n, l = x.shape
    s = 1
    while s < n:
        shifted = jnp.concatenate(
            [jnp.zeros((s, l), x.dtype), x[: n - s, :]], axis=0)
        x = x + shifted
        s *= 2
    return x


def _route_body(x_ref, rw_ref, rb_ref, p_ref, eid_ref):
    xb = x_ref[...].astype(jnp.bfloat16)

    logits = _nt(xb, rw_ref[...].astype(jnp.bfloat16)) + rb_ref[...]
    iota = lax.broadcasted_iota(jnp.int32, logits.shape, 1)
    m0 = jnp.max(logits, axis=1, keepdims=True)
    i0 = jnp.min(jnp.where(logits == m0, iota, NUM_E), axis=1, keepdims=True)
    oh0 = iota == i0
    masked = jnp.where(oh0, -jnp.inf, logits)
    m1 = jnp.max(masked, axis=1, keepdims=True)
    i1 = jnp.min(jnp.where(masked == m1, iota, NUM_E), axis=1, keepdims=True)
    oh1 = iota == i1
    t = jnp.exp(m1 - m0)
    w0 = 1.0 / (1.0 + t)
    w1 = t / (1.0 + t)

    mask = (oh0 | oh1).astype(jnp.float32)
    cum = _cumsum0(mask)
    rank_excl = cum - mask
    counts = cum[N_TOK - 1:N_TOK, :]
    padded = jnp.floor((counts + (BLK_T - 1)) * (1.0 / BLK_T)) * BLK_T
    r8 = lax.broadcasted_iota(jnp.int32, (NUM_E, NUM_E), 0)
    c8 = lax.broadcasted_iota(jnp.int32, (NUM_E, NUM_E), 1)
    tri = (r8 < c8).astype(jnp.float32)
    base = jax.lax.dot_general(padded, tri, (((1,), (0,)), ((), ())),
                               preferred_element_type=jnp.float32)

    base0 = jnp.sum(jnp.where(oh0, base, 0.0), axis=1, keepdims=True)
    base1 = jnp.sum(jnp.where(oh1, base, 0.0), axis=1, keepdims=True)
    r0 = jnp.sum(jnp.where(oh0, rank_excl, 0.0), axis=1, keepdims=True)
    r1 = jnp.sum(jnp.where(oh1, rank_excl, 0.0), axis=1, keepdims=True)
    p0 = base0 + r0
    p1 = base1 + r1
    lane = lax.broadcasted_iota(jnp.int32, (N_TOK, NUM_E), 1)
    p_ref[...] = (jnp.where(lane == 0, p0, 0.0) + jnp.where(lane == 1, p1, 0.0)
                  + jnp.where(lane == 2, w0, 0.0) + jnp.where(lane == 3, w1, 0.0))

    eye = (r8 == c8).astype(jnp.float32)
    start_col = _nt(eye, base * (1.0 / BLK_T))
    i64 = lax.broadcasted_iota(jnp.int32, (NUM_E, 64), 1).astype(jnp.float32)
    eid = jnp.sum((start_col <= i64).astype(jnp.float32), axis=0,
                  keepdims=True) - 1.0
    eid_ref[...] = eid.astype(jnp.int32)


def _route(flat, router_w, rb):
    return pl.pallas_call(
        _route_body,
        grid=(1,),
        in_specs=[
            pl.BlockSpec((N_TOK, D_MODEL), lambda i: (0, 0)),
            pl.BlockSpec((NUM_E, D_MODEL), lambda i: (0, 0)),
            pl.BlockSpec((1, NUM_E), lambda i: (0, 0)),
        ],
        out_specs=[
            pl.BlockSpec((N_TOK, NUM_E), lambda i: (0, 0)),
            pl.BlockSpec((1, 64), lambda i: (0, 0)),
        ],
        out_shape=[
            jax.ShapeDtypeStruct((N_TOK, NUM_E), jnp.float32),
            jax.ShapeDtypeStruct((1, 64), jnp.int32),
        ],
    )(flat, router_w, rb)


def _shared_body(x_ref, sg_ref, su_ref, sd_ref, sh_ref):
    xc = x_ref[...].astype(jnp.bfloat16)
    g = _nt(xc, sg_ref[...].astype(jnp.bfloat16))
    u = _nt(xc, su_ref[...].astype(jnp.bfloat16))
    h = ((g * jax.nn.sigmoid(g)) * u).astype(jnp.bfloat16)
    sh_ref[...] = _nt(h, sd_ref[...].astype(jnp.bfloat16)).astype(jnp.bfloat16)


def _shared(flat, sg, su, sd):
    tb = 1024
    return pl.pallas_call(
        _shared_body,
        grid=(N_TOK // tb,),
        in_specs=[
            pl.BlockSpec((tb, D_MODEL), lambda i: (i, 0)),
            pl.BlockSpec((INTER, D_MODEL), lambda i: (0, 0)),
            pl.BlockSpec((INTER, D_MODEL), lambda i: (0, 0)),
            pl.BlockSpec((D_MODEL, INTER), lambda i: (0, 0)),
        ],
        out_specs=pl.BlockSpec((tb, D_MODEL), lambda i: (i, 0)),
        out_shape=jax.ShapeDtypeStruct((N_TOK, D_MODEL), jnp.bfloat16),
    )(flat, sg, su, sd)


def _sc_scatter_body(x_hbm, p01_hbm, xg_hbm, idx_v, rows_v, sem):
    wid = lax.axis_index("s") * 2 + lax.axis_index("c")
    for j in range(4):
        abase = wid * 256 + j * CHUNK
        tbase = lax.rem(abase, N_TOK)
        pltpu.sync_copy(p01_hbm.at[pl.ds(abase, CHUNK)], idx_v)
        pltpu.sync_copy(x_hbm.at[pl.ds(tbase, CHUNK)], rows_v)
        pltpu.async_copy(rows_v, xg_hbm.at[idx_v], sem).wait()


def _sc_scatter(flat, p01):
    mesh = plsc.VectorSubcoreMesh(core_axis_name="c", subcore_axis_name="s")
    fn = functools.partial(
        pl.kernel,
        out_type=jax.ShapeDtypeStruct((XG_ROWS, D_MODEL), jnp.float32),
        mesh=mesh,
        scratch_types=[
            pltpu.VMEM((CHUNK,), jnp.int32),
            pltpu.VMEM((CHUNK, D_MODEL), jnp.float32),
            pltpu.SemaphoreType.DMA,
        ],
    )(_sc_scatter_body)
    return fn(flat, p01)


def _gemm_body(eid_ref, xg_ref, gw_ref, uw_ref, dw_ref, yg_ref):
    del eid_ref
    xb = xg_ref[...].astype(jnp.bfloat16)
    g = _nt(xb, gw_ref[0].astype(jnp.bfloat16))
    u = _nt(xb, uw_ref[0].astype(jnp.bfloat16))
    h = ((g * jax.nn.sigmoid(g)) * u).astype(jnp.bfloat16)
    yg_ref[...] = _nt(h, dw_ref[0].astype(jnp.bfloat16))


def _grouped_gemm(xg, eid, gate_ws, up_ws, down_ws):
    grid_spec = pltpu.PrefetchScalarGridSpec(
        num_scalar_prefetch=1,
        grid=(N_BLK,),
        in_specs=[
            pl.BlockSpec((BLK_T, D_MODEL), lambda i, eid: (i, 0)),
            pl.BlockSpec((1, INTER, D_MODEL), lambda i, eid: (eid[i], 0, 0)),
            pl.BlockSpec((1, INTER, D_MODEL), lambda i, eid: (eid[i], 0, 0)),
            pl.BlockSpec((1, D_MODEL, INTER), lambda i, eid: (eid[i], 0, 0)),
        ],
        out_specs=pl.BlockSpec((BLK_T, D_MODEL), lambda i, eid: (i, 0)),
    )
    return pl.pallas_call(
        _gemm_body,
        grid_spec=grid_spec,
        out_shape=jax.ShapeDtypeStruct((XG_ROWS, D_MODEL), jnp.float32),
    )(eid, xg, gate_ws, up_ws, down_ws)


def _sc_gather_body(yg_hbm, p01_hbm, a01_hbm, idx_v, rows_v, sem):
    wid = lax.axis_index("s") * 2 + lax.axis_index("c")
    for j in range(4):
        abase = wid * 256 + j * CHUNK
        pltpu.sync_copy(p01_hbm.at[pl.ds(abase, CHUNK)], idx_v)
        pltpu.async_copy(yg_hbm.at[idx_v], rows_v, sem).wait()
        pltpu.sync_copy(rows_v, a01_hbm.at[pl.ds(abase, CHUNK)])


def _sc_gather(yg, p01):
    mesh = plsc.VectorSubcoreMesh(core_axis_name="c", subcore_axis_name="s")
    fn = functools.partial(
        pl.kernel,
        out_type=jax.ShapeDtypeStruct((2 * N_TOK, D_MODEL), jnp.float32),
        mesh=mesh,
        scratch_types=[
            pltpu.VMEM((CHUNK,), jnp.int32),
            pltpu.VMEM((CHUNK, D_MODEL), jnp.float32),
            pltpu.SemaphoreType.DMA,
        ],
    )(_sc_gather_body)
    return fn(yg, p01)


def _combine_body(sh_ref, a0_ref, a1_ref, p_ref, out_ref):
    lane = lax.broadcasted_iota(jnp.int32, p_ref.shape, 1)
    w0 = jnp.sum(jnp.where(lane == 2, p_ref[...], 0.0), axis=1, keepdims=True)
    w1 = jnp.sum(jnp.where(lane == 3, p_ref[...], 0.0), axis=1, keepdims=True)
    out_ref[...] = (sh_ref[...].astype(jnp.float32)
                    + w0 * a0_ref[...] + w1 * a1_ref[...])


def _combine(sh, a01, p):
    tb = 1024
    return pl.pallas_call(
        _combine_body,
        grid=(N_TOK // tb,),
        in_specs=[
            pl.BlockSpec((tb, D_MODEL), lambda i: (i, 0)),
            pl.BlockSpec((tb, D_MODEL), lambda i: (i, 0)),
            pl.BlockSpec((tb, D_MODEL), lambda i: (i + N_TOK // tb, 0)),
            pl.BlockSpec((tb, NUM_E), lambda i: (i, 0)),
        ],
        out_specs=pl.BlockSpec((tb, D_MODEL), lambda i: (i, 0)),
        out_shape=jax.ShapeDtypeStruct((N_TOK, D_MODEL), jnp.float32),
    )(sh, a01, a01, p)


def kernel(x, router_w, router_bias, shared_gate_w, shared_up_w,
           shared_down_w, gate_ws, up_ws, down_ws):
    b, s, d = x.shape
    flat = x.reshape(b * s, d)
    rb = router_bias.reshape(1, NUM_E)

    p, eid = _route(flat, router_w, rb)
    p01 = jnp.concatenate([p[:, 0], p[:, 1]]).astype(jnp.int32)
    xg = _sc_scatter(flat, p01)
    sh = _shared(flat, shared_gate_w, shared_up_w, shared_down_w)
    yg = _grouped_gemm(xg, eid.reshape(64), gate_ws, up_ws, down_ws)
    a01 = _sc_gather(yg, p01)
    out = _combine(sh, a01, p)
    return out.reshape(b, s, d)


# speedup vs baseline: 1.4088x; 1.2744x over previous
import jax
import jax.numpy as jnp
from jax.experimental import pallas as pl
from jax.experimental.pallas import tpu as pltpu

D_MODEL = 1024
INTER = 512
NUM_E = 8
TOKEN_BLOCK = 1024


def _nt(a, b):
    return jax.lax.dot_general(
        a, b, (((1,), (1,)), ((), ())), preferred_element_type=jnp.float32
    )


def _moe_body(x_ref, rw_ref, rb_ref, sg_ref, su_ref, sd_ref,
              gw_ref, uw_ref, dw_ref, out_ref, wscr_ref):
    e = pl.program_id(1)
    xf = x_ref[...]
    xb = xf.astype(jnp.bfloat16)
    ne = rw_ref.shape[0]

    @pl.when(e == 0)
    def _init():
        logits = _nt(xb, rw_ref[...].astype(jnp.bfloat16)) + rb_ref[...]
        iota = jax.lax.broadcasted_iota(jnp.int32, logits.shape, 1)
        m0 = jnp.max(logits, axis=1, keepdims=True)
        i0 = jnp.min(jnp.where(logits == m0, iota, ne), axis=1, keepdims=True)
        oh0 = iota == i0
        masked = jnp.where(oh0, -jnp.inf, logits)
        m1 = jnp.max(masked, axis=1, keepdims=True)
        i1 = jnp.min(jnp.where(masked == m1, iota, ne), axis=1, keepdims=True)
        oh1 = iota == i1
        t = jnp.exp(m1 - m0)
        w0 = 1.0 / (1.0 + t)
        w1 = t / (1.0 + t)
        wscr_ref[...] = jnp.where(oh0, w0, 0.0) + jnp.where(oh1, w1, 0.0)

        g = _nt(xb, sg_ref[...].astype(jnp.bfloat16))
        u = _nt(xb, su_ref[...].astype(jnp.bfloat16))
        h = ((g * jax.nn.sigmoid(g)) * u).astype(jnp.bfloat16)
        out_ref[...] = _nt(h, sd_ref[...].astype(jnp.bfloat16))

    lane = jax.lax.broadcasted_iota(jnp.int32, wscr_ref.shape, 1)
    w_e = jnp.sum(jnp.where(lane == e, wscr_ref[...], 0.0), axis=1,
                  keepdims=True)
    g = _nt(xb, gw_ref[0].astype(jnp.bfloat16))
    u = _nt(xb, uw_ref[0].astype(jnp.bfloat16))
    h = (((g * jax.nn.sigmoid(g)) * u) * w_e).astype(jnp.bfloat16)
    out_ref[...] += _nt(h, dw_ref[0].astype(jnp.bfloat16))


def kernel(x, router_w, router_bias, shared_gate_w, shared_up_w,
           shared_down_w, gate_ws, up_ws, down_ws):
    b, s, d = x.shape
    n = b * s
    flat = x.reshape(n, d)
    rb = router_bias.reshape(1, NUM_E)
    grid = (n // TOKEN_BLOCK, NUM_E)
    out = pl.pallas_call(
        _moe_body,
        grid=grid,
        in_specs=[
            pl.BlockSpec((TOKEN_BLOCK, d), lambda t, e: (t, 0)),
            pl.BlockSpec((NUM_E, d), lambda t, e: (0, 0)),
            pl.BlockSpec((1, NUM_E), lambda t, e: (0, 0)),
            pl.BlockSpec((INTER, d), lambda t, e: (0, 0)),
            pl.BlockSpec((INTER, d), lambda t, e: (0, 0)),
            pl.BlockSpec((d, INTER), lambda t, e: (0, 0)),
            pl.BlockSpec((1, INTER, d), lambda t, e: (e, 0, 0)),
            pl.BlockSpec((1, INTER, d), lambda t, e: (e, 0, 0)),
            pl.BlockSpec((1, d, INTER), lambda t, e: (e, 0, 0)),
        ],
        out_specs=pl.BlockSpec((TOKEN_BLOCK, d), lambda t, e: (t, 0)),
        out_shape=jax.ShapeDtypeStruct((n, d), jnp.float32),
        scratch_shapes=[pltpu.VMEM((TOKEN_BLOCK, NUM_E), jnp.float32)],
        compiler_params=pltpu.CompilerParams(
            dimension_semantics=("parallel", "arbitrary")),
    )(flat, router_w, rb, shared_gate_w, shared_up_w, shared_down_w,
      gate_ws, up_ws, down_ws)
    return out.reshape(b, s, d)
